# Initial kernel scaffold; baseline (speedup 1.0000x reference)
import jax
import jax.numpy as jnp
from jax.experimental import pallas as pl


def kernel(x):
    raise NotImplementedError("write your pallas kernel here")


# speedup vs baseline: 10.8970x; 10.8970x over previous
import jax
import jax.numpy as jnp
from jax.experimental import pallas as pl
from jax.experimental.pallas import tpu as pltpu

_B, _C, _N = 8, 3, 2048
_R = 256
_INF = float("inf")


def _knn_means_body(x_ref, p_ref, s_ref, q_ref, f8_ref, f16_ref, f32_ref):
    xb = x_ref[0]
    ptab = p_ref[0]
    stab = s_ref[0]
    q = q_ref[0]

    inner = -2.0 * jnp.dot(q, xb, preferred_element_type=jnp.float32)
    xxj = xb[0:1, :] * xb[0:1, :] + xb[1:2, :] * xb[1:2, :] + xb[2:3, :] * xb[2:3, :]
    xxi = q[:, 0:1] * q[:, 0:1] + q[:, 1:2] * q[:, 1:2] + q[:, 2:3] * q[:, 2:3]
    s = ((0.0 - xxj) - inner) - xxi

    def extract(v):
        mx = jnp.max(v, axis=1, keepdims=True)
        return jnp.where(v == mx, -_INF, v)

    for _ in range(8):
        s = extract(s)
    m8 = (s == -_INF).astype(jnp.float32)
    hi = jax.lax.Precision.HIGHEST
    f8_ref[0] = jnp.dot(m8, ptab, precision=hi,
                        preferred_element_type=jnp.float32) * (1.0 / 8.0)
    for _ in range(8):
        s = extract(s)
    m16 = (s == -_INF).astype(jnp.float32)
    f16_ref[0] = jnp.dot(m16, stab, precision=hi,
                         preferred_element_type=jnp.float32) * (1.0 / 16.0)
    for _ in range(16):
        s = extract(s)
    m32 = (s == -_INF).astype(jnp.float32)
    f32_ref[0] = jnp.dot(m32, ptab, precision=hi,
                         preferred_element_type=jnp.float32) * (1.0 / 32.0)


def _knn_means(x0, ptab, stab):
    nb = _N // _R
    grid = (_B, nb)
    out_shape = [jax.ShapeDtypeStruct((_B, _N, 3), jnp.float32)] * 3
    in_specs = [
        pl.BlockSpec((1, _C, _N), lambda b, r: (b, 0, 0)),
        pl.BlockSpec((1, _N, 3), lambda b, r: (b, 0, 0)),
        pl.BlockSpec((1, _N, 3), lambda b, r: (b, 0, 0)),
        pl.BlockSpec((1, _R, 3), lambda b, r: (b, r, 0)),
    ]
    out_specs = [pl.BlockSpec((1, _R, 3), lambda b, r: (b, r, 0))] * 3
    return pl.pallas_call(
        _knn_means_body,
        grid=grid,
        in_specs=in_specs,
        out_specs=out_specs,
        out_shape=out_shape,
    )(x0, ptab, stab, ptab)


def kernel(x):
    batch_size = x.shape[0]
    num_points = x.shape[3]
    x0 = jnp.reshape(x, (batch_size, -1, num_points))
    ptab = jnp.swapaxes(x0, 1, 2)
    stab = jnp.reshape(x0, (batch_size, num_points, 3))
    f8, f16, f32 = _knn_means(x0, ptab, stab)

    concat_x = jnp.swapaxes(jnp.expand_dims(x0, 3), 2, 1)
    for feat in (f8, f16, f32):
        feature = feat.reshape(batch_size, num_points, 1, 1, 3)
        num_dims = concat_x.shape[3]
        concat_x = jnp.reshape(concat_x, (batch_size, num_points, 1, num_dims, 3))
        concat_x = jnp.concatenate((feature, concat_x), axis=3)
        concat_x = jnp.transpose(concat_x, (0, 4, 1, 3, 2))
    concat_x = jnp.transpose(concat_x, (0, 3, 1, 2, 4))
    return concat_x
